# baseline (speedup 1.0000x reference)
import functools

import jax
import jax.numpy as jnp
from jax.experimental import pallas as pl


_TS = 128
_CAND = 64
_TOPK = 32


def _select_top64(dist, scores):
    ts, n = dist.shape
    iota = jax.lax.broadcasted_iota(jnp.int32, (ts, n), 1)
    io64 = jax.lax.broadcasted_iota(jnp.int32, (ts, _CAND), 1)
    big = jnp.int32(n)

    def body(k, carry):
        d, ss, sd = carry
        m = jnp.min(d, axis=1, keepdims=True)
        j = jnp.min(jnp.where(d == m, iota, big), axis=1, keepdims=True)
        oh = iota == j
        sk = jnp.sum(jnp.where(oh, scores, 0.0), axis=1, keepdims=True)
        oh64 = io64 == k
        ss = ss + jnp.where(oh64, sk, 0.0)
        sd = sd + jnp.where(oh64, m, 0.0)
        d = jnp.where(oh, jnp.float32(jnp.inf), d)
        return d, ss, sd

    init = (
        dist,
        jnp.zeros((ts, _CAND), jnp.float32),
        jnp.zeros((ts, _CAND), jnp.float32),
    )
    _, sel_s, sel_d = jax.lax.fori_loop(0, _CAND, body, init, unroll=8)
    return sel_s, sel_d


def _threshold_gate(sel_s, tau):
    raw = sel_s - tau
    g = jnp.where(raw > 0, raw, jnp.float32(1e-8) * jnp.exp(raw))
    eg = jnp.exp(g) - 1.0
    ts = eg.shape[0]
    io = jax.lax.broadcasted_iota(jnp.int32, (ts, _CAND), 1)
    big = jnp.int32(_CAND)

    def body(i, carry):
        w, _ = carry
        cm = jnp.max(w, axis=1, keepdims=True)
        j = jnp.min(jnp.where(w == cm, io, big), axis=1, keepdims=True)
        w = jnp.where(io == j, -jnp.inf, w)
        return w, cm

    _, thr = jax.lax.fori_loop(
        0, _TOPK, body, (eg, jnp.zeros((ts, 1), jnp.float32))
    )
    keep = jnp.where(eg >= thr, eg, 0.0)
    gsum = jnp.sum(keep, axis=1, keepdims=True) + jnp.float32(1e-8)
    strength = jnp.tanh(jnp.max(keep, axis=1, keepdims=True))
    return keep / gsum * strength


def _pool_body(n_tau, x_ref, tab_ref, nposT_ref, pos_ref, tau_ref,
               *out_refs):
    gate_refs = out_refs[:n_tau]
    loss_ref = out_refs[n_tau]

    hi = jax.lax.Precision.HIGHEST
    xb = x_ref[...]
    xd = xb / jnp.float32(0.9)
    scores = jax.lax.dot_general(
        xd, tab_ref[...], (((1,), (1,)), ((), ())), precision=hi
    )
    pos = pos_ref[...]
    dx = pos[:, 0:1] - nposT_ref[0:1, :]
    dy = pos[:, 1:2] - nposT_ref[1:2, :]
    dist = dx * dx + dy * dy
    tau = tau_ref[...]

    sel_s, sel_d = _select_top64(dist, scores)

    gate0 = None
    for t in range(n_tau):
        gate_t = _threshold_gate(sel_s, tau[:, t:t + 1])
        gate_refs[t][...] = gate_t
        if t == 0:
            gate0 = gate_t

    partial = jnp.sum(gate0 * sel_d).reshape(1, 1)
    step = pl.program_id(0)
    prev = jnp.where(step == 0, jnp.zeros_like(partial), loss_ref[...])
    loss_ref[...] = prev + partial


def _pool_call(x2d, table, nposT, pos, tau):
    s, d = x2d.shape
    n = table.shape[0]
    n_tau = tau.shape[1]
    grid = (s // _TS,)
    kernel_fn = functools.partial(_pool_body, n_tau)
    out_shape = (
        [jax.ShapeDtypeStruct((s, _CAND), jnp.float32) for _ in range(n_tau)]
        + [jax.ShapeDtypeStruct((1, 1), jnp.float32)]
    )
    in_specs = [
        pl.BlockSpec((_TS, d), lambda i: (i, 0)),
        pl.BlockSpec((n, d), lambda i: (0, 0)),
        pl.BlockSpec((2, n), lambda i: (0, 0)),
        pl.BlockSpec((_TS, 2), lambda i: (i, 0)),
        pl.BlockSpec((_TS, n_tau), lambda i: (i, 0)),
    ]
    out_specs = (
        [pl.BlockSpec((_TS, _CAND), lambda i: (i, 0)) for _ in range(n_tau)]
        + [pl.BlockSpec((1, 1), lambda i: (0, 0))]
    )
    return pl.pallas_call(
        kernel_fn,
        grid=grid,
        in_specs=in_specs,
        out_specs=out_specs,
        out_shape=out_shape,
    )(x2d, table, nposT, pos, tau)


def kernel(x, qk_neurons, v_neurons, know_neurons, neuron_pos, W_pos_qk,
           b_pos_qk, W_pos_v, b_pos_v, W_pos_know, b_pos_know, W_tau_attn,
           b_tau_attn, W_tau_know, b_tau_know):
    b, s, d = x.shape
    n_qk = qk_neurons.shape[0]
    n_v = v_neurons.shape[0]
    x2d = x.reshape(b * s, d)

    qk_pos = (x @ W_pos_qk + b_pos_qk).reshape(b * s, -1)
    v_pos = (x @ W_pos_v + b_pos_v).reshape(b * s, -1)
    know_pos = (x @ W_pos_know + b_pos_know).reshape(b * s, -1)
    tau_all = (x @ W_tau_attn + b_tau_attn).reshape(b * s, -1)
    tau_kn = (x @ W_tau_know + b_tau_know).reshape(b * s, -1)

    npos_qk = neuron_pos[:n_qk].T
    npos_v = neuron_pos[n_qk:n_qk + n_v].T
    npos_kn = neuron_pos[n_qk + n_v:].T

    gq, gk, loss_qk = _pool_call(
        x2d, qk_neurons, npos_qk, qk_pos, tau_all[:, 0:2])
    gv, loss_v = _pool_call(
        x2d, v_neurons, npos_v, v_pos, tau_all[:, 2:3])
    gn, loss_kn = _pool_call(
        x2d, know_neurons, npos_kn, know_pos, tau_kn)

    denom = jnp.float32(b * s * _CAND) + jnp.float32(1e-8)
    pos_loss_attn = (loss_qk[0, 0] + loss_v[0, 0]) / denom
    pos_loss_know = loss_kn[0, 0] / denom

    return (
        gq.reshape(b, s, _CAND),
        gk.reshape(b, s, _CAND),
        gv.reshape(b, s, _CAND),
        gn.reshape(b, s, _CAND),
        pos_loss_attn,
        pos_loss_know,
    )


# speedup vs baseline: 1.2970x; 1.2970x over previous
import functools

import jax
import jax.numpy as jnp
from jax.experimental import pallas as pl


_TS = 256
_CAND = 64
_TOPK = 32


def _select_top64(dist, scores):
    ts, n = dist.shape
    iota = jax.lax.broadcasted_iota(jnp.int32, (ts, n), 1)
    io64 = jax.lax.broadcasted_iota(jnp.int32, (ts, _CAND), 1)
    big = jnp.int32(n)

    def body(k, carry):
        d, ss, sd = carry
        m = jnp.min(d, axis=1, keepdims=True)
        j = jnp.min(jnp.where(d == m, iota, big), axis=1, keepdims=True)
        oh = iota == j
        sk = jnp.sum(jnp.where(oh, scores, 0.0), axis=1, keepdims=True)
        oh64 = io64 == k
        ss = ss + jnp.where(oh64, sk, 0.0)
        sd = sd + jnp.where(oh64, m, 0.0)
        d = jnp.where(oh, jnp.float32(jnp.inf), d)
        return d, ss, sd

    init = (
        dist,
        jnp.zeros((ts, _CAND), jnp.float32),
        jnp.zeros((ts, _CAND), jnp.float32),
    )
    _, sel_s, sel_d = jax.lax.fori_loop(0, _CAND, body, init, unroll=8)
    return sel_s, sel_d


def _threshold_gate(sel_s, tau):
    raw = sel_s - tau
    g = jnp.where(raw > 0, raw, jnp.float32(1e-8) * jnp.exp(raw))
    eg = jnp.exp(g) - 1.0
    ts = eg.shape[0]
    io = jax.lax.broadcasted_iota(jnp.int32, (ts, _CAND), 1)
    big = jnp.int32(_CAND)

    def body(i, carry):
        w, _ = carry
        cm = jnp.max(w, axis=1, keepdims=True)
        j = jnp.min(jnp.where(w == cm, io, big), axis=1, keepdims=True)
        w = jnp.where(io == j, -jnp.inf, w)
        return w, cm

    _, thr = jax.lax.fori_loop(
        0, _TOPK, body, (eg, jnp.zeros((ts, 1), jnp.float32))
    )
    keep = jnp.where(eg >= thr, eg, 0.0)
    gsum = jnp.sum(keep, axis=1, keepdims=True) + jnp.float32(1e-8)
    strength = jnp.tanh(jnp.max(keep, axis=1, keepdims=True))
    return keep / gsum * strength


def _pool_body(n_tau, x_ref, tab_ref, nposT_ref, pos_ref, tau_ref,
               *out_refs):
    gate_refs = out_refs[:n_tau]
    loss_ref = out_refs[n_tau]

    hi = jax.lax.Precision.HIGHEST
    xb = x_ref[...]
    xd = xb / jnp.float32(0.9)
    scores = jax.lax.dot_general(
        xd, tab_ref[...], (((1,), (1,)), ((), ())), precision=hi
    )
    pos = pos_ref[...]
    dx = pos[:, 0:1] - nposT_ref[0:1, :]
    dy = pos[:, 1:2] - nposT_ref[1:2, :]
    dist = dx * dx + dy * dy
    tau = tau_ref[...]

    sel_s, sel_d = _select_top64(dist, scores)

    gate0 = None
    for t in range(n_tau):
        gate_t = _threshold_gate(sel_s, tau[:, t:t + 1])
        gate_refs[t][...] = gate_t
        if t == 0:
            gate0 = gate_t

    partial = jnp.sum(gate0 * sel_d).reshape(1, 1)
    step = pl.program_id(0)
    prev = jnp.where(step == 0, jnp.zeros_like(partial), loss_ref[...])
    loss_ref[...] = prev + partial


def _pool_call(x2d, table, nposT, pos, tau):
    s, d = x2d.shape
    n = table.shape[0]
    n_tau = tau.shape[1]
    grid = (s // _TS,)
    kernel_fn = functools.partial(_pool_body, n_tau)
    out_shape = (
        [jax.ShapeDtypeStruct((s, _CAND), jnp.float32) for _ in range(n_tau)]
        + [jax.ShapeDtypeStruct((1, 1), jnp.float32)]
    )
    in_specs = [
        pl.BlockSpec((_TS, d), lambda i: (i, 0)),
        pl.BlockSpec((n, d), lambda i: (0, 0)),
        pl.BlockSpec((2, n), lambda i: (0, 0)),
        pl.BlockSpec((_TS, 2), lambda i: (i, 0)),
        pl.BlockSpec((_TS, n_tau), lambda i: (i, 0)),
    ]
    out_specs = (
        [pl.BlockSpec((_TS, _CAND), lambda i: (i, 0)) for _ in range(n_tau)]
        + [pl.BlockSpec((1, 1), lambda i: (0, 0))]
    )
    return pl.pallas_call(
        kernel_fn,
        grid=grid,
        in_specs=in_specs,
        out_specs=out_specs,
        out_shape=out_shape,
    )(x2d, table, nposT, pos, tau)


def kernel(x, qk_neurons, v_neurons, know_neurons, neuron_pos, W_pos_qk,
           b_pos_qk, W_pos_v, b_pos_v, W_pos_know, b_pos_know, W_tau_attn,
           b_tau_attn, W_tau_know, b_tau_know):
    b, s, d = x.shape
    n_qk = qk_neurons.shape[0]
    n_v = v_neurons.shape[0]
    x2d = x.reshape(b * s, d)

    qk_pos = (x @ W_pos_qk + b_pos_qk).reshape(b * s, -1)
    v_pos = (x @ W_pos_v + b_pos_v).reshape(b * s, -1)
    know_pos = (x @ W_pos_know + b_pos_know).reshape(b * s, -1)
    tau_all = (x @ W_tau_attn + b_tau_attn).reshape(b * s, -1)
    tau_kn = (x @ W_tau_know + b_tau_know).reshape(b * s, -1)

    npos_qk = neuron_pos[:n_qk].T
    npos_v = neuron_pos[n_qk:n_qk + n_v].T
    npos_kn = neuron_pos[n_qk + n_v:].T

    gq, gk, loss_qk = _pool_call(
        x2d, qk_neurons, npos_qk, qk_pos, tau_all[:, 0:2])
    gv, loss_v = _pool_call(
        x2d, v_neurons, npos_v, v_pos, tau_all[:, 2:3])
    gn, loss_kn = _pool_call(
        x2d, know_neurons, npos_kn, know_pos, tau_kn)

    denom = jnp.float32(b * s * _CAND) + jnp.float32(1e-8)
    pos_loss_attn = (loss_qk[0, 0] + loss_v[0, 0]) / denom
    pos_loss_know = loss_kn[0, 0] / denom

    return (
        gq.reshape(b, s, _CAND),
        gk.reshape(b, s, _CAND),
        gv.reshape(b, s, _CAND),
        gn.reshape(b, s, _CAND),
        pos_loss_attn,
        pos_loss_know,
    )
